# baseline (speedup 1.0000x reference)
import jax
import jax.numpy as jnp
from jax import lax
from jax.experimental import pallas as pl
from jax.experimental.pallas import tpu as pltpu


_HB = 32


def _body(added_ref, orig_ref, boxes_ref, sbox_ref, out_ref, ia_ref):
    h = pl.program_id(1)
    HB = added_ref.shape[2]
    W = added_ref.shape[3]

    pred = jnp.sum(added_ref[0], axis=0)
    orig = jnp.sum(orig_ref[0, 1:], axis=0)
    occ_p = jnp.abs(pred) > 0.0
    occ_o = jnp.abs(orig) > 0.0
    occ_and = jnp.logical_and(occ_p, occ_o)
    occ_or = jnp.logical_or(occ_p, occ_o)

    row = lax.broadcasted_iota(jnp.int32, (HB, W), 0) + h * HB
    col = lax.broadcasted_iota(jnp.int32, (HB, W), 1)
    x = row.astype(jnp.float32) * 0.8
    y = col.astype(jnp.float32) * 0.8

    bx = boxes_ref[0]
    M = bx.shape[0]
    c = jnp.cos(bx[:, 6:7])
    s = jnp.sin(bx[:, 6:7])

    ia_ref[...] = jnp.zeros((HB, W), jnp.float32)

    x_lo = (h * HB).astype(jnp.float32) * 0.8
    x_hi = x_lo + (HB - 1) * 0.8
    for m in range(M):
        cxs = sbox_ref[0, m, 0]
        czs = sbox_ref[0, m, 2]
        dxs = sbox_ref[0, m, 3]
        dys = sbox_ref[0, m, 4]
        dzs = sbox_ref[0, m, 5]
        rmax = 0.5 * (dxs + dys)
        hit_rows = jnp.logical_and(cxs + rmax > x_lo, cxs - rmax < x_hi)
        in_z = jnp.abs(czs) < dzs * 0.5
        live = jnp.logical_and(hit_rows, in_z)

        @pl.when(live)
        def _():
            sx = x - bx[m : m + 1, 0:1]
            sy = y - bx[m : m + 1, 1:2]
            cm = c[m : m + 1, 0:1]
            sm = s[m : m + 1, 0:1]
            lx = sx * cm + sy * sm
            ly = sy * cm - sx * sm
            hit = jnp.logical_and(
                jnp.abs(lx) < bx[m : m + 1, 3:4] * 0.5,
                jnp.abs(ly) < bx[m : m + 1, 4:5] * 0.5,
            )
            ia_ref[...] = jnp.maximum(ia_ref[...], hit.astype(jnp.float32))

    in_any = ia_ref[...] > 0.0
    inter = jnp.sum(jnp.where(jnp.logical_and(in_any, occ_and), 1.0, 0.0))
    union = jnp.sum(jnp.where(jnp.logical_and(in_any, occ_or), 1.0, 0.0))

    lane = lax.broadcasted_iota(jnp.int32, (1, 1, 128), 2)
    v = jnp.where(lane == 0, inter, 0.0) + jnp.where(lane == 1, union, 0.0)

    @pl.when(h == 0)
    def _():
        out_ref[...] = v

    @pl.when(h != 0)
    def _():
        out_ref[...] += v


def kernel(added_points, original_points, boxes):
    B, C, H, W = added_points.shape
    M = boxes.shape[1]
    nh = H // _HB

    out = pl.pallas_call(
        _body,
        grid=(B, nh),
        in_specs=[
            pl.BlockSpec((1, C, _HB, W), lambda b, h: (b, 0, h, 0)),
            pl.BlockSpec((1, C + 1, _HB, W), lambda b, h: (b, 0, h, 0)),
            pl.BlockSpec((1, M, 7), lambda b, h: (b, 0, 0)),
            pl.BlockSpec(memory_space=pltpu.SMEM, block_shape=(1, M, 7),
                         index_map=lambda b, h: (b, 0, 0)),
        ],
        out_specs=pl.BlockSpec((1, 1, 128), lambda b, h: (b, 0, 0)),
        out_shape=jax.ShapeDtypeStruct((B, 1, 128), jnp.float32),
        scratch_shapes=[pltpu.VMEM((_HB, W), jnp.float32)],
    )(added_points, original_points, boxes, boxes)

    inter = out[:, 0, 0]
    union = out[:, 0, 1]
    return jnp.mean(M * inter / (union + 1e-6))


# speedup vs baseline: 1.9725x; 1.9725x over previous
import jax
import jax.numpy as jnp
from jax import lax
from jax.experimental import pallas as pl
from jax.experimental.pallas import tpu as pltpu


_HB = 64


def _body(added_ref, orig_ref, boxes_ref, boxesT_ref, out_ref, acc_ref):
    h = pl.program_id(1)
    nh = pl.num_programs(1)
    HB = added_ref.shape[2]
    W = added_ref.shape[3]

    pred = jnp.sum(added_ref[0], axis=0)
    orig = jnp.sum(orig_ref[0, 1:], axis=0)
    occ_p = jnp.abs(pred) > 0.0
    occ_o = jnp.abs(orig) > 0.0
    occ_and = jnp.logical_and(occ_p, occ_o)
    occ_or = jnp.logical_or(occ_p, occ_o)

    bT = boxesT_ref[0]
    bC = boxes_ref[0]
    M = bC.shape[0]

    c_r = jnp.cos(bT[6:7, :])
    s_r = jnp.sin(bT[6:7, :])
    zok_r = jnp.abs(bT[2:3, :]) < bT[5:6, :] * 0.5
    iex_r = 2.0 / bT[3:4, :]
    iey_r = 2.0 / bT[4:5, :]
    tx_r = jnp.where(zok_r, -(bT[0:1, :] * c_r + bT[1:2, :] * s_r) * iex_r, 1e9)
    ty_r = jnp.where(zok_r, (bT[0:1, :] * s_r - bT[1:2, :] * c_r) * iey_r, 1e9)

    c_c = jnp.cos(bC[:, 6:7])
    s_c = jnp.sin(bC[:, 6:7])
    iex_c = 2.0 / bC[:, 3:4]
    iey_c = 2.0 / bC[:, 4:5]

    xcol = (lax.broadcasted_iota(jnp.int32, (HB, 1), 0) + h * HB
            ).astype(jnp.float32) * 0.8
    U1 = xcol * (c_r * iex_r) + tx_r
    U2 = xcol * (-s_r * iey_r) + ty_r

    yrow = lax.broadcasted_iota(jnp.int32, (1, W), 1).astype(jnp.float32) * 0.8
    V1 = (s_c * iex_c) * yrow
    V2 = (c_c * iey_c) * yrow

    score = None
    for m in range(M):
        lx = U1[:, m : m + 1] + V1[m : m + 1, :]
        ly = U2[:, m : m + 1] + V2[m : m + 1, :]
        d = jnp.maximum(jnp.abs(lx), jnp.abs(ly))
        score = d if score is None else jnp.minimum(score, d)
    in_any = score < 1.0

    w_i = jnp.where(jnp.logical_and(in_any, occ_and), 1.0, 0.0)
    w_u = jnp.where(jnp.logical_and(in_any, occ_or), 1.0, 0.0)
    fi = jnp.zeros((8, 128), jnp.float32)
    fu = jnp.zeros((8, 128), jnp.float32)
    for r in range(HB // 8):
        for cc in range(W // 128):
            fi = fi + w_i[8 * r : 8 * r + 8, 128 * cc : 128 * cc + 128]
            fu = fu + w_u[8 * r : 8 * r + 8, 128 * cc : 128 * cc + 128]

    @pl.when(h == 0)
    def _():
        acc_ref[0] = fi
        acc_ref[1] = fu

    @pl.when(h != 0)
    def _():
        acc_ref[0] += fi
        acc_ref[1] += fu

    @pl.when(h == nh - 1)
    def _():
        inter = jnp.sum(acc_ref[0])
        union = jnp.sum(acc_ref[1])
        lane = lax.broadcasted_iota(jnp.int32, (1, 1, 128), 2)
        out_ref[...] = (jnp.where(lane == 0, inter, 0.0)
                        + jnp.where(lane == 1, union, 0.0))


def kernel(added_points, original_points, boxes):
    B, C, H, W = added_points.shape
    M = boxes.shape[1]
    nh = H // _HB
    boxesT = jnp.transpose(boxes, (0, 2, 1))

    out = pl.pallas_call(
        _body,
        grid=(B, nh),
        in_specs=[
            pl.BlockSpec((1, C, _HB, W), lambda b, h: (b, 0, h, 0)),
            pl.BlockSpec((1, C + 1, _HB, W), lambda b, h: (b, 0, h, 0)),
            pl.BlockSpec((1, M, 7), lambda b, h: (b, 0, 0)),
            pl.BlockSpec((1, 7, M), lambda b, h: (b, 0, 0)),
        ],
        out_specs=pl.BlockSpec((1, 1, 128), lambda b, h: (b, 0, 0)),
        out_shape=jax.ShapeDtypeStruct((B, 1, 128), jnp.float32),
        scratch_shapes=[pltpu.VMEM((2, 8, 128), jnp.float32)],
        compiler_params=pltpu.CompilerParams(
            dimension_semantics=("parallel", "arbitrary")),
    )(added_points, original_points, boxes, boxesT)

    inter = out[:, 0, 0]
    union = out[:, 0, 1]
    return jnp.mean(M * inter / (union + 1e-6))


# speedup vs baseline: 2.6866x; 1.3621x over previous
import jax
import jax.numpy as jnp
from jax import lax
from jax.experimental import pallas as pl
from jax.experimental.pallas import tpu as pltpu


def _body(added_ref, orig_ref, boxes_ref, boxesT_ref, out_ref):
    H = added_ref.shape[2]
    W = added_ref.shape[3]

    pred = jnp.sum(added_ref[0], axis=0)
    orig = jnp.sum(orig_ref[0, 1:], axis=0)
    occ_p = jnp.abs(pred) > 0.0
    occ_o = jnp.abs(orig) > 0.0
    occ_and = jnp.logical_and(occ_p, occ_o)
    occ_or = jnp.logical_or(occ_p, occ_o)

    bT = boxesT_ref[0]
    bC = boxes_ref[0]
    M = bC.shape[0]

    c_r = jnp.cos(bT[6:7, :])
    s_r = jnp.sin(bT[6:7, :])
    zok_r = jnp.abs(bT[2:3, :]) < bT[5:6, :] * 0.5
    iex_r = 2.0 / bT[3:4, :]
    iey_r = 2.0 / bT[4:5, :]
    tx_r = jnp.where(zok_r, -(bT[0:1, :] * c_r + bT[1:2, :] * s_r) * iex_r, 1e9)
    ty_r = jnp.where(zok_r, (bT[0:1, :] * s_r - bT[1:2, :] * c_r) * iey_r, 1e9)

    c_c = jnp.cos(bC[:, 6:7])
    s_c = jnp.sin(bC[:, 6:7])
    iex_c = 2.0 / bC[:, 3:4]
    iey_c = 2.0 / bC[:, 4:5]

    xcol = lax.broadcasted_iota(jnp.int32, (H, 1), 0).astype(jnp.float32) * 0.8
    U1 = xcol * (c_r * iex_r) + tx_r
    U2 = xcol * (-s_r * iey_r) + ty_r

    yrow = lax.broadcasted_iota(jnp.int32, (1, W), 1).astype(jnp.float32) * 0.8
    V1 = (s_c * iex_c) * yrow
    V2 = (c_c * iey_c) * yrow

    score = None
    for m in range(M):
        lx = U1[:, m : m + 1] + V1[m : m + 1, :]
        ly = U2[:, m : m + 1] + V2[m : m + 1, :]
        d = jnp.maximum(jnp.abs(lx), jnp.abs(ly))
        score = d if score is None else jnp.minimum(score, d)
    in_any = score < 1.0

    w_i = jnp.where(jnp.logical_and(in_any, occ_and), 1.0, 0.0)
    w_u = jnp.where(jnp.logical_and(in_any, occ_or), 1.0, 0.0)
    fi = jnp.zeros((8, 128), jnp.float32)
    fu = jnp.zeros((8, 128), jnp.float32)
    for r in range(H // 8):
        for cc in range(W // 128):
            fi = fi + w_i[8 * r : 8 * r + 8, 128 * cc : 128 * cc + 128]
            fu = fu + w_u[8 * r : 8 * r + 8, 128 * cc : 128 * cc + 128]

    inter = jnp.sum(fi)
    union = jnp.sum(fu)
    lane = lax.broadcasted_iota(jnp.int32, (1, 1, 128), 2)
    out_ref[...] = (jnp.where(lane == 0, inter, 0.0)
                    + jnp.where(lane == 1, union, 0.0))


def kernel(added_points, original_points, boxes):
    B, C, H, W = added_points.shape
    M = boxes.shape[1]
    boxesT = jnp.transpose(boxes, (0, 2, 1))

    out = pl.pallas_call(
        _body,
        grid=(B,),
        in_specs=[
            pl.BlockSpec((1, C, H, W), lambda b: (b, 0, 0, 0)),
            pl.BlockSpec((1, C + 1, H, W), lambda b: (b, 0, 0, 0)),
            pl.BlockSpec((1, M, 7), lambda b: (b, 0, 0)),
            pl.BlockSpec((1, 7, M), lambda b: (b, 0, 0)),
        ],
        out_specs=pl.BlockSpec((1, 1, 128), lambda b: (b, 0, 0)),
        out_shape=jax.ShapeDtypeStruct((B, 1, 128), jnp.float32),
        compiler_params=pltpu.CompilerParams(
            dimension_semantics=("arbitrary",)),
    )(added_points, original_points, boxes, boxesT)

    inter = out[:, 0, 0]
    union = out[:, 0, 1]
    return jnp.mean(M * inter / (union + 1e-6))
